# baseline (speedup 1.0000x reference)
import jax
import jax.numpy as jnp
from jax.experimental import pallas as pl

_BATCH_BLK = 8


def _add_body(x_ref, t_ref, o_ref):
    o_ref[...] = x_ref[...] + t_ref[...]


def kernel(patches, pos_table):
    b, n, d = patches.shape
    flat = n * d
    x = patches.reshape(b, flat)
    t = pos_table.reshape(1, flat)
    grid = (b // _BATCH_BLK,)
    out = pl.pallas_call(
        _add_body,
        grid=grid,
        in_specs=[
            pl.BlockSpec((_BATCH_BLK, flat), lambda i: (i, 0)),
            pl.BlockSpec((1, flat), lambda i: (0, 0)),
        ],
        out_specs=pl.BlockSpec((_BATCH_BLK, flat), lambda i: (i, 0)),
        out_shape=jax.ShapeDtypeStruct((b, flat), patches.dtype),
    )(x, t)
    return out.reshape(b, n, d)


# speedup vs baseline: 1.0125x; 1.0125x over previous
import jax
import jax.numpy as jnp
from jax.experimental import pallas as pl

_BATCH_BLK = 32


def _add_body(x_ref, t_ref, o_ref):
    o_ref[...] = x_ref[...] + t_ref[...]


def kernel(patches, pos_table):
    b, n, d = patches.shape
    flat = n * d
    x = patches.reshape(b, flat)
    t = pos_table.reshape(1, flat)
    grid = (b // _BATCH_BLK,)
    out = pl.pallas_call(
        _add_body,
        grid=grid,
        in_specs=[
            pl.BlockSpec((_BATCH_BLK, flat), lambda i: (i, 0)),
            pl.BlockSpec((1, flat), lambda i: (0, 0)),
        ],
        out_specs=pl.BlockSpec((_BATCH_BLK, flat), lambda i: (i, 0)),
        out_shape=jax.ShapeDtypeStruct((b, flat), patches.dtype),
    )(x, t)
    return out.reshape(b, n, d)


# speedup vs baseline: 1.3230x; 1.3067x over previous
import jax
import jax.numpy as jnp
from jax.experimental import pallas as pl

_BATCH_BLK = 8


def _add_body(x_ref, t_ref, o_ref):
    o_ref[...] = x_ref[...] + t_ref[...]


def kernel(patches, pos_table):
    b, n, d = patches.shape
    grid = (b // _BATCH_BLK,)
    return pl.pallas_call(
        _add_body,
        grid=grid,
        in_specs=[
            pl.BlockSpec((_BATCH_BLK, n, d), lambda i: (i, 0, 0)),
            pl.BlockSpec((1, n, d), lambda i: (0, 0, 0)),
        ],
        out_specs=pl.BlockSpec((_BATCH_BLK, n, d), lambda i: (i, 0, 0)),
        out_shape=jax.ShapeDtypeStruct((b, n, d), patches.dtype),
    )(patches, pos_table.reshape(1, n, d))
